# baseline (speedup 1.0000x reference)
import functools

import jax
import jax.numpy as jnp
from jax import lax
from jax.experimental import pallas as pl
from jax.experimental.pallas import tpu as pltpu
from jax.experimental.pallas import tpu_sc as plsc

VOCAB = 100000
DIM = 128
BATCH = 4096
SEQ = 50

NC = 2
NS = 16
NW = NC * NS
NB_W = BATCH // NW
G = 4
GIDX = G * SEQ
GSTRIDE = 256
NG = NB_W // G
MG = 2
NSLOT = 2 * MG

_mesh = plsc.VectorSubcoreMesh(core_axis_name="c", subcore_axis_name="s")


@functools.partial(
    pl.kernel,
    mesh=_mesh,
    out_type=jax.ShapeDtypeStruct((BATCH, SEQ, DIM), jnp.float32),
    scratch_types=[
        pltpu.VMEM((NG * GSTRIDE,), jnp.int32),
        pltpu.VMEM((NSLOT, GIDX, DIM), jnp.float32),
        pltpu.SemaphoreType.DMA,
        pltpu.SemaphoreType.DMA,
    ],
    compiler_params=pltpu.CompilerParams(
        use_tc_tiling_on_sc=True, needs_layout_passes=True
    ),
)
def _embed(ids_hbm, table_hbm, out_hbm, idx_v, slots, gsem, ssem):
    wid = lax.axis_index("s") * NC + lax.axis_index("c")
    bbase = wid * NB_W
    pltpu.sync_copy(ids_hbm.at[pl.ds(wid * NG * GSTRIDE, NG * GSTRIDE)], idx_v)

    def gather_group(g, b):
        off = pl.multiple_of(g * GSTRIDE, 8)
        pltpu.async_copy(
            table_hbm.at[idx_v.at[pl.ds(off, GIDX)]], slots.at[b], gsem
        )

    def wait_gather_group(b):
        pltpu.make_async_copy(
            table_hbm.at[idx_v.at[pl.ds(0, GIDX)]], slots.at[b], gsem
        ).wait()

    def scatter_group(g, b):
        for k in range(G):
            pltpu.async_copy(
                slots.at[b, pl.ds(k * SEQ, SEQ)], out_hbm.at[bbase + g * G + k], ssem
            )

    def wait_scatter():
        for k in range(G):
            pltpu.make_async_copy(
                slots.at[0, pl.ds(0, SEQ)], out_hbm.at[bbase], ssem
            ).wait()

    for b in range(MG):
        gather_group(b, b)
    for g in range(MG):
        wait_gather_group(g)
        scatter_group(g, g)
        gather_group(g + MG, (g + MG) % NSLOT)
    def body(g, carry):
        b = lax.rem(g, NSLOT)
        wait_gather_group(b)
        scatter_group(g, b)
        wait_scatter()
        gather_group(g + MG, lax.rem(g + MG, NSLOT))
        return carry

    lax.fori_loop(MG, NG - MG, body, 0)
    for g in range(NG - MG, NG):
        wait_gather_group(g % NSLOT)
        scatter_group(g, g % NSLOT)
    for _ in range(NSLOT):
        wait_scatter()


def kernel(token_ids, table):
    ids = token_ids.astype(jnp.int32).reshape(BATCH * SEQ // GIDX, GIDX)
    ids = jnp.pad(ids, ((0, 0), (0, GSTRIDE - GIDX)))
    return _embed(ids.reshape(-1), table)


# speedup vs baseline: 1.8095x; 1.8095x over previous
import functools

import jax
import jax.numpy as jnp
from jax import lax
from jax.experimental import pallas as pl
from jax.experimental.pallas import tpu as pltpu
from jax.experimental.pallas import tpu_sc as plsc

VOCAB = 100000
DIM = 128
BATCH = 4096
SEQ = 50

NC = 2
NS = 16
NW = NC * NS
NB_W = BATCH // NW
M = 3
NBUF = 2 * M

_mesh = plsc.VectorSubcoreMesh(core_axis_name="c", subcore_axis_name="s")


@functools.partial(
    pl.kernel,
    mesh=_mesh,
    out_type=jax.ShapeDtypeStruct((SEQ, BATCH, DIM), jnp.float32),
    scratch_types=[
        pltpu.VMEM((SEQ, NB_W), jnp.int32),
        pltpu.VMEM((NBUF, NB_W, DIM), jnp.float32),
        pltpu.SemaphoreType.DMA,
        pltpu.SemaphoreType.DMA,
    ],
)
def _embed(ids_hbm, table_hbm, out_hbm, idx_v, bufs, gsem, ssem):
    wid = lax.axis_index("s") * NC + lax.axis_index("c")
    bbase = wid * NB_W
    pltpu.sync_copy(ids_hbm.at[pl.ds(0, SEQ), wid], idx_v)

    def gather(g, b):
        pltpu.async_copy(table_hbm.at[idx_v.at[g]], bufs.at[b], gsem)

    def scatter(g, b):
        pltpu.async_copy(bufs.at[b], out_hbm.at[g, pl.ds(bbase, NB_W)], ssem)

    def wait_gather(b):
        pltpu.make_async_copy(table_hbm.at[idx_v.at[0]], bufs.at[b], gsem).wait()

    def wait_scatter():
        pltpu.make_async_copy(bufs.at[0], out_hbm.at[0, pl.ds(bbase, NB_W)], ssem).wait()

    for b in range(M):
        gather(b, b)
    for g in range(M):
        wait_gather(g)
        scatter(g, g)
        gather(g + M, (g + M) % NBUF)
    def body(g, carry):
        b = lax.rem(g, NBUF)
        wait_gather(b)
        scatter(g, b)
        wait_scatter()
        gather(g + M, lax.rem(g + M, NBUF))
        return carry

    lax.fori_loop(M, SEQ - M, body, 0)
    for g in range(SEQ - M, SEQ):
        wait_gather(g % NBUF)
        scatter(g, g % NBUF)
    for _ in range(NBUF):
        wait_scatter()


def kernel(token_ids, table):
    ids_t = token_ids.astype(jnp.int32).T.reshape(SEQ, NW, NB_W)
    out = _embed(ids_t, table)
    return jnp.transpose(out, (1, 0, 2))
